# baseline (speedup 1.0000x reference)
import functools

import jax
import jax.numpy as jnp
from jax import lax
from jax.experimental import pallas as pl
from jax.experimental.pallas import tpu as pltpu
from jax.experimental.pallas import tpu_sc as plsc

EMBED_DIM = 32
NUM_CORES = 2
NUM_SUBCORES = 16
NUM_WORKERS = NUM_CORES * NUM_SUBCORES

ROWS_PER_CHUNK = 4
NBUF = 4
LA = NBUF - 1


@functools.lru_cache(maxsize=None)
def _make_gather(n_rows, n_cols):
    idx_per_chunk = ROWS_PER_CHUNK * n_cols
    rows_per_w = n_rows // NUM_WORKERS
    n_chunks = rows_per_w // ROWS_PER_CHUNK
    n_grp = n_chunks // NBUF
    assert n_rows % NUM_WORKERS == 0
    assert rows_per_w % ROWS_PER_CHUNK == 0 and n_chunks % NBUF == 0
    mesh = plsc.VectorSubcoreMesh(core_axis_name="c", subcore_axis_name="s")

    scratch = (
        [pltpu.VMEM((idx_per_chunk,), jnp.int32) for _ in range(NBUF)]
        + [pltpu.VMEM((idx_per_chunk, EMBED_DIM), jnp.float32)
           for _ in range(NBUF)]
        + [pltpu.SemaphoreType.DMA for _ in range(2 * NBUF)]
    )

    @functools.partial(
        pl.kernel,
        mesh=mesh,
        compiler_params=pltpu.CompilerParams(use_tc_tiling_on_sc=False),
        out_type=jax.ShapeDtypeStruct((n_rows, n_cols, EMBED_DIM),
                                      jnp.float32),
        scratch_types=scratch,
    )
    def gather_kernel(x_hbm, table_hbm, out_hbm, *sc):
        idx_bufs = sc[0:NBUF]
        row_bufs = sc[NBUF:2 * NBUF]
        gsem = sc[2 * NBUF:3 * NBUF]
        wsem = sc[3 * NBUF:4 * NBUF]
        wid = lax.axis_index("s") * NUM_CORES + lax.axis_index("c")
        base = wid * rows_per_w

        def launch(chunk, b):
            r0 = base + chunk * ROWS_PER_CHUNK
            for i in range(ROWS_PER_CHUNK):
                pltpu.sync_copy(x_hbm.at[r0 + i],
                                idx_bufs[b].at[pl.ds(i * n_cols, n_cols)])
            pltpu.async_copy(
                table_hbm.at[idx_bufs[b]], row_bufs[b], gsem[b])

        def gather_wait(b):
            pltpu.make_async_copy(
                table_hbm.at[idx_bufs[b]], row_bufs[b], gsem[b]).wait()

        def wb_start(chunk, b):
            r0 = base + chunk * ROWS_PER_CHUNK
            for i in range(ROWS_PER_CHUNK):
                pltpu.async_copy(
                    row_bufs[b].at[pl.ds(i * n_cols, n_cols)],
                    out_hbm.at[r0 + i], wsem[b])

        def wb_wait(b):
            for i in range(ROWS_PER_CHUNK):
                pltpu.make_async_copy(
                    row_bufs[b].at[pl.ds(i * n_cols, n_cols)],
                    out_hbm.at[base + i], wsem[b]).wait()

        for t in range(LA):
            launch(t, t % NBUF)

        @pl.loop(0, n_grp)
        def _(grp):
            for b in range(NBUF):
                c = grp * NBUF + b
                bg = (b + LA) % NBUF

                if b == 0:
                    @pl.when(grp > 0)
                    def _():
                        wb_wait(bg)
                    launch(c + LA, bg)
                else:
                    @pl.when(grp < n_grp - 1)
                    def _():
                        wb_wait(bg)
                        launch(c + LA, bg)

                gather_wait(b)
                wb_start(c, b)

        for b in range(NBUF):
            wb_wait(b)

    return gather_kernel


def kernel(x, tool_embeddings):
    return _make_gather(x.shape[0], x.shape[1])(x, tool_embeddings)
